# baseline (speedup 1.0000x reference)
import functools

import jax
import jax.numpy as jnp
from jax import lax
from jax.experimental import pallas as pl
from jax.experimental.pallas import tpu as pltpu
from jax.experimental.pallas import tpu_sc as plsc

M = 4096
IN_DIM = 384
D = 512
K = 6
KPAD = 8
BLK = 256
NBLK = M // BLK
NEG = -1.0e30
SCALE = D ** -0.5


def _leaky(v):
    return jnp.where(v >= 0, v, v * 0.01)


def _mm(a, b):
    return lax.dot_general(a, b, (((1,), (0,)), ((), ())),
                           preferred_element_type=jnp.float32)


def _front_body(x_ref, w1_ref, b1_ref, wh_ref, bh_ref, wt_ref, bt_ref,
                eh_ref, etp_ref, idx_ref, h0s, cs, ehbs, etbs):
    p = pl.program_id(0)
    i = pl.program_id(1)
    rows = pl.ds(i * BLK, BLK)

    @pl.when(p == 0)
    def _():
        h = _leaky(_mm(x_ref[...], w1_ref[...]) + b1_ref[...])
        h0s[rows, :] = h

        @pl.when(i == 0)
        def _():
            cs[...] = jnp.zeros_like(cs)

        cs[...] += jnp.sum(h, axis=0, keepdims=True)

    @pl.when(p == 1)
    def _():
        h0 = (h0s[rows, :] + cs[...] * (1.0 / M)) * 0.5
        eh = _mm(h0, wh_ref[...]) + bh_ref[...]
        et = _mm(h0, wt_ref[...]) + bt_ref[...]
        eh_ref[...] = eh
        ehbs[rows, :] = eh.astype(jnp.bfloat16)
        etbs[rows, :] = et.astype(jnp.bfloat16)
        lo = lax.bitcast_convert_type(et[:, :D // 2], jnp.int32) + 0x8000
        hi = lax.bitcast_convert_type(et[:, D // 2:], jnp.int32) + 0x8000
        etp_ref[...] = ((lo >> 16) & 0xFFFF) | (hi & jnp.int32(-65536))

    @pl.when(p == 2)
    def _():
        logits = lax.dot_general(etbs[...], ehbs[rows, :],
                                 (((1,), (1,)), ((), ())),
                                 preferred_element_type=jnp.float32)
        bits = lax.bitcast_convert_type(logits, jnp.int32)
        mono = bits ^ ((bits >> 31) & jnp.int32(0x7FFFFFFF))
        rrow = lax.broadcasted_iota(jnp.int32, (M, BLK), 0) ^ jnp.int32(0xFFF)
        key = (mono & jnp.int32(-4096)) | rrow
        sub = lax.broadcasted_iota(jnp.int32, (KPAD, BLK), 0)
        acc = jnp.zeros((KPAD, BLK), jnp.int32)
        for k in range(K):
            part = jnp.max(key.reshape(32, M // 32, BLK), axis=0)
            mk = jnp.max(part, axis=0, keepdims=True)
            idx_k = (mk & jnp.int32(0xFFF)) ^ jnp.int32(0xFFF)
            acc = jnp.where(sub == k, idx_k, acc)
            key = jnp.where(key == mk, jnp.int32(-0x80000000), key)
        idx_ref[...] = acc


def _run_front(x2, fc1_W, fc1_b2, Wh_W, Wh_b2, Wt_W, Wt_b2):
    return pl.pallas_call(
        _front_body,
        grid=(3, NBLK),
        in_specs=[
            pl.BlockSpec((BLK, IN_DIM),
                         lambda p, i: (jnp.where(p == 0, i, NBLK - 1), 0)),
            pl.BlockSpec((IN_DIM, D), lambda p, i: (0, 0)),
            pl.BlockSpec((1, D), lambda p, i: (0, 0)),
            pl.BlockSpec((D, D), lambda p, i: (0, 0)),
            pl.BlockSpec((1, D), lambda p, i: (0, 0)),
            pl.BlockSpec((D, D), lambda p, i: (0, 0)),
            pl.BlockSpec((1, D), lambda p, i: (0, 0)),
        ],
        out_specs=[
            pl.BlockSpec((BLK, D), lambda p, i: (
                jnp.where(p == 1, i, jnp.where(p == 0, 0, NBLK - 1)), 0)),
            pl.BlockSpec((BLK, D // 2), lambda p, i: (
                jnp.where(p == 1, i, jnp.where(p == 0, 0, NBLK - 1)), 0)),
            pl.BlockSpec((KPAD, BLK),
                         lambda p, i: (0, jnp.where(p == 2, i, 0))),
        ],
        out_shape=[
            jax.ShapeDtypeStruct((M, D), jnp.float32),
            jax.ShapeDtypeStruct((M, D // 2), jnp.int32),
            jax.ShapeDtypeStruct((KPAD, M), jnp.int32),
        ],
        scratch_shapes=[
            pltpu.VMEM((M, D), jnp.float32),
            pltpu.VMEM((1, D), jnp.float32),
            pltpu.VMEM((M, D), jnp.bfloat16),
            pltpu.VMEM((M, D), jnp.bfloat16),
        ],
    )(x2, fc1_W, fc1_b2, Wh_W, Wh_b2, Wt_W, Wt_b2)


_SC_NC = 2
_SC_NS = 16
_NROW = K * M
_PER_W = _NROW // (_SC_NC * _SC_NS)
_CHUNK = 128
_NCHUNK = _PER_W // _CHUNK
_NBUF = 3


def _gather_sc(table, idx3):
    mesh = plsc.VectorSubcoreMesh(core_axis_name="c", subcore_axis_name="s",
                                  num_cores=_SC_NC, num_subcores=_SC_NS)

    @functools.partial(
        pl.kernel,
        out_type=jax.ShapeDtypeStruct((_NROW, D // 2), jnp.int32),
        mesh=mesh,
        scratch_types=[
            pltpu.VMEM((_NCHUNK, _CHUNK), jnp.int32),
        ] + [pltpu.VMEM((_CHUNK, D // 2), jnp.int32)] * _NBUF
          + [pltpu.SemaphoreType.DMA] * (2 * _NBUF),
    )
    def gather_kernel(table_hbm, idx_hbm, out_hbm, idx_v, *bufs_sems):
        rows = bufs_sems[:_NBUF]
        gsem = bufs_sems[_NBUF:2 * _NBUF]
        wsem = bufs_sems[2 * _NBUF:]
        wid = lax.axis_index("s") * _SC_NC + lax.axis_index("c")
        base = wid * _PER_W
        pltpu.sync_copy(idx_hbm.at[wid], idx_v)
        gathers = [None] * _NCHUNK
        writes = [None] * _NCHUNK
        for c in range(_NBUF):
            gathers[c] = pltpu.async_copy(table_hbm.at[idx_v.at[c]],
                                          rows[c], gsem[c])
        for c in range(_NCHUNK):
            b = c % _NBUF
            gathers[c].wait()
            writes[c] = pltpu.async_copy(
                rows[b], out_hbm.at[pl.ds(base + c * _CHUNK, _CHUNK)],
                wsem[b])
            if c + _NBUF < _NCHUNK:
                writes[c].wait()
                gathers[c + _NBUF] = pltpu.async_copy(
                    table_hbm.at[idx_v.at[c + _NBUF]], rows[b], gsem[b])
        for c in range(_NCHUNK - _NBUF, _NCHUNK):
            writes[c].wait()

    return gather_kernel(table, idx3)


def _fuse_body(eh_ref, n0, n1, n2, n3, n4, n5,
               w1_ref, b1_ref, w2_ref, b2_ref, g_ref, bb_ref,
               out_ref, acc_ref):
    i = pl.program_id(0)
    eh = eh_ref[...]
    eh_l = eh[:, :D // 2]
    eh_h = eh[:, D // 2:]
    ns = []
    for r in (n0, n1, n2, n3, n4, n5):
        word = r[...]
        n_l = lax.bitcast_convert_type(word << 16, jnp.float32)
        n_h = lax.bitcast_convert_type(word & jnp.int32(-65536), jnp.float32)
        ns.append((n_l, n_h))

    w = [(jnp.sum(eh_l * nl + eh_h * nh, axis=1, keepdims=True)) * SCALE
         for nl, nh in ns]
    mx = w[0]
    for k in range(1, K):
        mx = jnp.maximum(mx, w[k])
    ew = [jnp.exp(wk - mx) for wk in w]
    z = ew[0]
    for k in range(1, K):
        z = z + ew[k]
    p = [e / z for e in ew]

    a = []
    for (nl, nh), pk in zip(ns, p):
        nsum = jnp.sum(nl + nh, axis=1, keepdims=True)
        gsum = jnp.sum(jnp.tanh(pk * nl + (2.0 - pk) * eh_l)
                       + jnp.tanh(pk * nh + (2.0 - pk) * eh_h),
                       axis=1, keepdims=True)
        a.append(nsum * gsum)
    mx2 = a[0]
    for k in range(1, K):
        mx2 = jnp.maximum(mx2, a[k])
    ea = [jnp.exp(ak - mx2) for ak in a]
    z2 = ea[0]
    for k in range(1, K):
        z2 = z2 + ea[k]

    q = [e / z2 for e in ea]
    e_nh_l = q[0] * ns[0][0]
    e_nh_h = q[0] * ns[0][1]
    for k in range(1, K):
        e_nh_l = e_nh_l + q[k] * ns[k][0]
        e_nh_h = e_nh_h + q[k] * ns[k][1]

    e_nh = jnp.concatenate([e_nh_l, e_nh_h], axis=1)
    emb = (_leaky(_mm(eh + e_nh, w1_ref[...]) + b1_ref[...])
           + _leaky(_mm(eh * e_nh, w2_ref[...]) + b2_ref[...]))

    @pl.when(i == 0)
    def _():
        acc_ref[...] = jnp.zeros_like(acc_ref)

    acc_ref[...] += jnp.sum(emb, axis=0, keepdims=True)

    @pl.when(i == NBLK - 1)
    def _():
        h = acc_ref[...] * (1.0 / M)
        mu = jnp.mean(h, axis=1, keepdims=True)
        var = jnp.mean((h - mu) ** 2, axis=1, keepdims=True)
        out_ref[...] = ((h - mu) * lax.rsqrt(var + 1e-5) * g_ref[...]
                        + bb_ref[...])


def _run_fuse(e_h, nb, lin1_W, lin1_b2, lin2_W, lin2_b2, ln_g2, ln_b2):
    def nb_spec(k):
        return pl.BlockSpec((BLK, D // 2), lambda i, k=k: (k * NBLK + i, 0))

    return pl.pallas_call(
        _fuse_body,
        grid=(NBLK,),
        in_specs=[
            pl.BlockSpec((BLK, D), lambda i: (i, 0)),
            nb_spec(0), nb_spec(1), nb_spec(2), nb_spec(3), nb_spec(4),
            nb_spec(5),
            pl.BlockSpec((D, D), lambda i: (0, 0)),
            pl.BlockSpec((1, D), lambda i: (0, 0)),
            pl.BlockSpec((D, D), lambda i: (0, 0)),
            pl.BlockSpec((1, D), lambda i: (0, 0)),
            pl.BlockSpec((1, D), lambda i: (0, 0)),
            pl.BlockSpec((1, D), lambda i: (0, 0)),
        ],
        out_specs=pl.BlockSpec((1, D), lambda i: (0, 0)),
        out_shape=jax.ShapeDtypeStruct((1, D), jnp.float32),
        scratch_shapes=[pltpu.VMEM((1, D), jnp.float32)],
    )(e_h, nb, nb, nb, nb, nb, nb,
      lin1_W, lin1_b2, lin2_W, lin2_b2, ln_g2, ln_b2)


def kernel(x, fc1_W, fc1_b, Wh_W, Wh_b, Wt_W, Wt_b,
           lin1_W, lin1_b, lin2_W, lin2_b, ln_g, ln_b):
    x2 = x.reshape(M, IN_DIM)
    fc1_b2 = fc1_b.reshape(1, D)
    Wh_b2 = Wh_b.reshape(1, D)
    Wt_b2 = Wt_b.reshape(1, D)
    lin1_b2 = lin1_b.reshape(1, D)
    lin2_b2 = lin2_b.reshape(1, D)
    ln_g2 = ln_g.reshape(1, D)
    ln_b2 = ln_b.reshape(1, D)

    e_h, e_tp, idx8 = _run_front(x2, fc1_W, fc1_b2, Wh_W, Wh_b2, Wt_W,
                                 Wt_b2)
    idx3 = idx8[:K].reshape(_SC_NC * _SC_NS, _NCHUNK, _CHUNK)
    nb = _gather_sc(e_tp, idx3)
    out = _run_fuse(e_h, nb, lin1_W, lin1_b2, lin2_W, lin2_b2, ln_g2, ln_b2)
    return out.reshape(D)


# speedup vs baseline: 1.0145x; 1.0145x over previous
import functools

import jax
import jax.numpy as jnp
from jax import lax
from jax.experimental import pallas as pl
from jax.experimental.pallas import tpu as pltpu
from jax.experimental.pallas import tpu_sc as plsc

M = 4096
IN_DIM = 384
D = 512
K = 6
KPAD = 8
BLK = 256
NBLK = M // BLK
NEG = -1.0e30
SCALE = D ** -0.5


def _leaky(v):
    return jnp.where(v >= 0, v, v * 0.01)


def _mm(a, b):
    return lax.dot_general(a, b, (((1,), (0,)), ((), ())),
                           preferred_element_type=jnp.float32)


def _front_body(x_ref, w1_ref, b1_ref, wh_ref, bh_ref, wt_ref, bt_ref,
                eh_ref, etp_ref, idx_ref, h0s, cs, ehbs, etbs):
    p = pl.program_id(0)
    i = pl.program_id(1)
    rows = pl.ds(i * BLK, BLK)

    @pl.when(p == 0)
    def _():
        h = _leaky(_mm(x_ref[...], w1_ref[...]) + b1_ref[...])
        h0s[rows, :] = h

        @pl.when(i == 0)
        def _():
            cs[...] = jnp.zeros_like(cs)

        cs[...] += jnp.sum(h, axis=0, keepdims=True)

    @pl.when(p == 1)
    def _():
        h0 = (h0s[rows, :] + cs[...] * (1.0 / M)) * 0.5
        eh = _mm(h0, wh_ref[...]) + bh_ref[...]
        et = _mm(h0, wt_ref[...]) + bt_ref[...]
        eh_ref[...] = eh
        ehbs[rows, :] = eh.astype(jnp.bfloat16)
        etbs[rows, :] = et.astype(jnp.bfloat16)
        lo = lax.bitcast_convert_type(et[:, :D // 2], jnp.int32) + 0x8000
        hi = lax.bitcast_convert_type(et[:, D // 2:], jnp.int32) + 0x8000
        etp_ref[...] = ((lo >> 16) & 0xFFFF) | (hi & jnp.int32(-65536))

    @pl.when(p == 2)
    def _():
        logits = lax.dot_general(etbs[...], ehbs[rows, :],
                                 (((1,), (1,)), ((), ())),
                                 preferred_element_type=jnp.float32)
        bits = lax.bitcast_convert_type(logits, jnp.int32)
        mono = bits ^ ((bits >> 31) & jnp.int32(0x7FFFFFFF))
        rrow = lax.broadcasted_iota(jnp.int32, (M, BLK), 0) ^ jnp.int32(0xFFF)
        key = (mono & jnp.int32(-4096)) | rrow
        sub = lax.broadcasted_iota(jnp.int32, (KPAD, BLK), 0)
        acc = jnp.zeros((KPAD, BLK), jnp.int32)
        mk = jnp.max(key, axis=0, keepdims=True)
        for k in range(K):
            idx_k = (mk & jnp.int32(0xFFF)) ^ jnp.int32(0xFFF)
            acc = jnp.where(sub == k, idx_k, acc)
            if k + 1 < K:
                mk = jnp.max(jnp.where(key < mk, key,
                                       jnp.int32(-0x80000000)),
                             axis=0, keepdims=True)
        idx_ref[...] = acc


def _run_front(x2, fc1_W, fc1_b2, Wh_W, Wh_b2, Wt_W, Wt_b2):
    return pl.pallas_call(
        _front_body,
        grid=(3, NBLK),
        in_specs=[
            pl.BlockSpec((BLK, IN_DIM),
                         lambda p, i: (jnp.where(p == 0, i, NBLK - 1), 0)),
            pl.BlockSpec((IN_DIM, D), lambda p, i: (0, 0)),
            pl.BlockSpec((1, D), lambda p, i: (0, 0)),
            pl.BlockSpec((D, D), lambda p, i: (0, 0)),
            pl.BlockSpec((1, D), lambda p, i: (0, 0)),
            pl.BlockSpec((D, D), lambda p, i: (0, 0)),
            pl.BlockSpec((1, D), lambda p, i: (0, 0)),
        ],
        out_specs=[
            pl.BlockSpec((BLK, D), lambda p, i: (
                jnp.where(p == 1, i, jnp.where(p == 0, 0, NBLK - 1)), 0)),
            pl.BlockSpec((BLK, D // 2), lambda p, i: (
                jnp.where(p == 1, i, jnp.where(p == 0, 0, NBLK - 1)), 0)),
            pl.BlockSpec((KPAD, BLK),
                         lambda p, i: (0, jnp.where(p == 2, i, 0))),
        ],
        out_shape=[
            jax.ShapeDtypeStruct((M, D), jnp.float32),
            jax.ShapeDtypeStruct((M, D // 2), jnp.int32),
            jax.ShapeDtypeStruct((KPAD, M), jnp.int32),
        ],
        scratch_shapes=[
            pltpu.VMEM((M, D), jnp.float32),
            pltpu.VMEM((1, D), jnp.float32),
            pltpu.VMEM((M, D), jnp.bfloat16),
            pltpu.VMEM((M, D), jnp.bfloat16),
        ],
    )(x2, fc1_W, fc1_b2, Wh_W, Wh_b2, Wt_W, Wt_b2)


_SC_NC = 2
_SC_NS = 16
_NROW = K * M
_PER_W = _NROW // (_SC_NC * _SC_NS)
_CHUNK = 128
_NCHUNK = _PER_W // _CHUNK
_NBUF = 3


def _gather_sc(table, idx3):
    mesh = plsc.VectorSubcoreMesh(core_axis_name="c", subcore_axis_name="s",
                                  num_cores=_SC_NC, num_subcores=_SC_NS)

    @functools.partial(
        pl.kernel,
        out_type=jax.ShapeDtypeStruct((_NROW, D // 2), jnp.int32),
        mesh=mesh,
        scratch_types=[
            pltpu.VMEM((_NCHUNK, _CHUNK), jnp.int32),
        ] + [pltpu.VMEM((_CHUNK, D // 2), jnp.int32)] * _NBUF
          + [pltpu.SemaphoreType.DMA] * (2 * _NBUF),
    )
    def gather_kernel(table_hbm, idx_hbm, out_hbm, idx_v, *bufs_sems):
        rows = bufs_sems[:_NBUF]
        gsem = bufs_sems[_NBUF:2 * _NBUF]
        wsem = bufs_sems[2 * _NBUF:]
        wid = lax.axis_index("s") * _SC_NC + lax.axis_index("c")
        base = wid * _PER_W
        pltpu.sync_copy(idx_hbm.at[wid], idx_v)
        gathers = [None] * _NCHUNK
        writes = [None] * _NCHUNK
        for c in range(_NBUF):
            gathers[c] = pltpu.async_copy(table_hbm.at[idx_v.at[c]],
                                          rows[c], gsem[c])
        for c in range(_NCHUNK):
            b = c % _NBUF
            gathers[c].wait()
            writes[c] = pltpu.async_copy(
                rows[b], out_hbm.at[pl.ds(base + c * _CHUNK, _CHUNK)],
                wsem[b])
            if c + _NBUF < _NCHUNK:
                writes[c].wait()
                gathers[c + _NBUF] = pltpu.async_copy(
                    table_hbm.at[idx_v.at[c + _NBUF]], rows[b], gsem[b])
        for c in range(_NCHUNK - _NBUF, _NCHUNK):
            writes[c].wait()

    return gather_kernel(table, idx3)


def _fuse_body(eh_ref, n0, n1, n2, n3, n4, n5,
               w1_ref, b1_ref, w2_ref, b2_ref, g_ref, bb_ref,
               out_ref, acc_ref):
    i = pl.program_id(0)
    eh = eh_ref[...]
    eh_l = eh[:, :D // 2]
    eh_h = eh[:, D // 2:]
    ns = []
    for r in (n0, n1, n2, n3, n4, n5):
        word = r[...]
        n_l = lax.bitcast_convert_type(word << 16, jnp.float32)
        n_h = lax.bitcast_convert_type(word & jnp.int32(-65536), jnp.float32)
        ns.append((n_l, n_h))

    w = [(jnp.sum(eh_l * nl + eh_h * nh, axis=1, keepdims=True)) * SCALE
         for nl, nh in ns]
    mx = w[0]
    for k in range(1, K):
        mx = jnp.maximum(mx, w[k])
    ew = [jnp.exp(wk - mx) for wk in w]
    z = ew[0]
    for k in range(1, K):
        z = z + ew[k]
    p = [e / z for e in ew]

    a = []
    for (nl, nh), pk in zip(ns, p):
        nsum = jnp.sum(nl + nh, axis=1, keepdims=True)
        gsum = jnp.sum(jnp.tanh(pk * nl + (2.0 - pk) * eh_l)
                       + jnp.tanh(pk * nh + (2.0 - pk) * eh_h),
                       axis=1, keepdims=True)
        a.append(nsum * gsum)
    mx2 = a[0]
    for k in range(1, K):
        mx2 = jnp.maximum(mx2, a[k])
    ea = [jnp.exp(ak - mx2) for ak in a]
    z2 = ea[0]
    for k in range(1, K):
        z2 = z2 + ea[k]

    q = [e / z2 for e in ea]
    e_nh_l = q[0] * ns[0][0]
    e_nh_h = q[0] * ns[0][1]
    for k in range(1, K):
        e_nh_l = e_nh_l + q[k] * ns[k][0]
        e_nh_h = e_nh_h + q[k] * ns[k][1]

    e_nh = jnp.concatenate([e_nh_l, e_nh_h], axis=1)
    emb = (_leaky(_mm(eh + e_nh, w1_ref[...]) + b1_ref[...])
           + _leaky(_mm(eh * e_nh, w2_ref[...]) + b2_ref[...]))

    @pl.when(i == 0)
    def _():
        acc_ref[...] = jnp.zeros_like(acc_ref)

    acc_ref[...] += jnp.sum(emb, axis=0, keepdims=True)

    @pl.when(i == NBLK - 1)
    def _():
        h = acc_ref[...] * (1.0 / M)
        mu = jnp.mean(h, axis=1, keepdims=True)
        var = jnp.mean((h - mu) ** 2, axis=1, keepdims=True)
        out_ref[...] = ((h - mu) * lax.rsqrt(var + 1e-5) * g_ref[...]
                        + bb_ref[...])


def _run_fuse(e_h, nb, lin1_W, lin1_b2, lin2_W, lin2_b2, ln_g2, ln_b2):
    def nb_spec(k):
        return pl.BlockSpec((BLK, D // 2), lambda i, k=k: (k * NBLK + i, 0))

    return pl.pallas_call(
        _fuse_body,
        grid=(NBLK,),
        in_specs=[
            pl.BlockSpec((BLK, D), lambda i: (i, 0)),
            nb_spec(0), nb_spec(1), nb_spec(2), nb_spec(3), nb_spec(4),
            nb_spec(5),
            pl.BlockSpec((D, D), lambda i: (0, 0)),
            pl.BlockSpec((1, D), lambda i: (0, 0)),
            pl.BlockSpec((D, D), lambda i: (0, 0)),
            pl.BlockSpec((1, D), lambda i: (0, 0)),
            pl.BlockSpec((1, D), lambda i: (0, 0)),
            pl.BlockSpec((1, D), lambda i: (0, 0)),
        ],
        out_specs=pl.BlockSpec((1, D), lambda i: (0, 0)),
        out_shape=jax.ShapeDtypeStruct((1, D), jnp.float32),
        scratch_shapes=[pltpu.VMEM((1, D), jnp.float32)],
    )(e_h, nb, nb, nb, nb, nb, nb,
      lin1_W, lin1_b2, lin2_W, lin2_b2, ln_g2, ln_b2)


def kernel(x, fc1_W, fc1_b, Wh_W, Wh_b, Wt_W, Wt_b,
           lin1_W, lin1_b, lin2_W, lin2_b, ln_g, ln_b):
    x2 = x.reshape(M, IN_DIM)
    fc1_b2 = fc1_b.reshape(1, D)
    Wh_b2 = Wh_b.reshape(1, D)
    Wt_b2 = Wt_b.reshape(1, D)
    lin1_b2 = lin1_b.reshape(1, D)
    lin2_b2 = lin2_b.reshape(1, D)
    ln_g2 = ln_g.reshape(1, D)
    ln_b2 = ln_b.reshape(1, D)

    e_h, e_tp, idx8 = _run_front(x2, fc1_W, fc1_b2, Wh_W, Wh_b2, Wt_W,
                                 Wt_b2)
    idx3 = idx8[:K].reshape(_SC_NC * _SC_NS, _NCHUNK, _CHUNK)
    nb = _gather_sc(e_tp, idx3)
    out = _run_fuse(e_h, nb, lin1_W, lin1_b2, lin2_W, lin2_b2, ln_g2, ln_b2)
    return out.reshape(D)
